# baseline (speedup 1.0000x reference)
import functools

import jax
import jax.numpy as jnp
from jax import lax
from jax.experimental import pallas as pl
from jax.experimental.pallas import tpu as pltpu
from jax.experimental.pallas import tpu_sc as plsc

N = 10000
D = 128
H = 64

NC = 2
NS = 16
NW = NC * NS
CH = 128
CPT = 80
NCHUNK = NW * CPT
E_PAD = NCHUNK * CH
N_ACC = 10112
RPT = N_ACC // NS
NBUF = 2
SHIFT = 14
MASK = (1 << SHIFT) - 1
L = 16

_vmesh = plsc.VectorSubcoreMesh(core_axis_name="c", subcore_axis_name="s")


@functools.partial(
    pl.kernel,
    out_type=jax.ShapeDtypeStruct((NC, N_ACC, D), jnp.float32),
    mesh=_vmesh,
    scratch_types=[
        pltpu.VMEM((CPT, CH), jnp.int32),
        pltpu.VMEM((NBUF, CH), jnp.int32),
        pltpu.VMEM((NBUF, CH), jnp.int32),
        pltpu.VMEM((NBUF, CH, D), jnp.float32),
        pltpu.VMEM_SHARED((N_ACC, D), jnp.float32),
        pltpu.SemaphoreType.DMA((NBUF,)),
        pltpu.SemaphoreType.DMA((NBUF,)),
    ],
)
def _sc_segsum(table_hbm, epk_hbm, out_hbm,
               epk, sidx, didx, bufs, acc, gsem, ssem):
    c = lax.axis_index("c")
    s = lax.axis_index("s")
    base_chunk = (c * NS + s) * CPT

    pltpu.sync_copy(epk_hbm.at[pl.ds(base_chunk, CPT)], epk)

    @pl.loop(0, CH)
    def _(i):
        @pl.loop(0, D // L)
        def _(j):
            bufs[0, i, pl.ds(j * L, L)] = jnp.zeros((L,), jnp.float32)

    @pl.loop(0, RPT // CH)
    def _(j):
        pltpu.sync_copy(bufs.at[0], acc.at[pl.ds(s * RPT + j * CH, CH)])

    pltpu.sync_copy(bufs.at[0, pl.ds(0, RPT % CH)],
                    acc.at[pl.ds(s * RPT + (RPT // CH) * CH, RPT % CH)])

    plsc.subcore_barrier()

    def _unpack(j):
        b = j % NBUF
        for g in range(CH // L):
            v = epk[j, pl.ds(g * L, L)]
            sidx[b, pl.ds(g * L, L)] = jnp.bitwise_and(v, MASK)
            didx[b, pl.ds(g * L, L)] = jnp.right_shift(v, SHIFT)

    def _gather(j):
        b = j % NBUF
        return pltpu.make_async_copy(table_hbm.at[sidx.at[b]], bufs.at[b],
                                     gsem.at[b])

    def _scatter(i):
        b = i % NBUF
        return pltpu.make_async_copy(bufs.at[b], acc.at[didx.at[b]],
                                     ssem.at[b])

    _unpack(0)
    pltpu.async_copy(table_hbm.at[sidx.at[0]], bufs.at[0], gsem.at[0])

    for i in range(CPT):
        b = i % NBUF
        _gather(i).wait()
        pltpu.async_copy(bufs.at[b], acc.at[didx.at[b]], ssem.at[b],
                         add=True)
        j = i + 1
        if j < CPT:
            jb = j % NBUF
            if j >= NBUF:
                _scatter(j - NBUF).wait()
            _unpack(j)
            pltpu.async_copy(table_hbm.at[sidx.at[jb]], bufs.at[jb],
                             gsem.at[jb])

    for m in range(CPT - NBUF, CPT):
        _scatter(m).wait()

    plsc.subcore_barrier()

    pltpu.sync_copy(acc.at[pl.ds(s * RPT, RPT)],
                    out_hbm.at[c].at[pl.ds(s * RPT, RPT)])


_BM = 1000
_GRID = N // _BM


def _k1_body(x_ref, w_ref, b_ref, t_ref, xr_ref):
    o = jnp.dot(x_ref[...], w_ref[...], preferred_element_type=jnp.float32)
    t_ref[:, :H] = o[:, :H]
    t_ref[:, H:] = jnp.zeros((_BM, H), jnp.float32)
    t_ref[:, H:H + 1] = jnp.ones((_BM, 1), jnp.float32)
    xr_ref[...] = o[:, H:] + b_ref[...]


def _k2_body(p_ref, xr_ref, w_ref, b_ref, t_ref, xe_ref):
    psum = p_ref[0] + p_ref[1]
    invd = 1.0 / jnp.maximum(psum[:, H:H + 1], 1.0)
    h = jnp.maximum(psum[:, :H] * invd + xr_ref[...], 0.0)
    o = jnp.dot(h, w_ref[...], preferred_element_type=jnp.float32)
    t_ref[:, :H] = o[:, :H]
    t_ref[:, H:] = jnp.zeros((_BM, H), jnp.float32)
    xe_ref[:, :H] = o[:, H:] + b_ref[...]
    xe_ref[:, H:] = jnp.zeros((_BM, H), jnp.float32)
    xe_ref[:, H:H + 1] = invd


def _k3_body(q_ref, xe_ref, wrow_ref, b_ref, out_ref):
    qsum = q_ref[0] + q_ref[1]
    invd = xe_ref[:, H:H + 1]
    h = jnp.maximum(qsum[:, :H] * invd + xe_ref[:, :H], 0.0)
    out_ref[...] = (jnp.sum(h * wrow_ref[...], axis=1, keepdims=True)
                    + b_ref[...])


_full = lambda *shape: pl.BlockSpec(shape, lambda m: tuple(0 for _ in shape))

_k1 = pl.pallas_call(
    _k1_body,
    grid=(_GRID,),
    in_specs=[
        pl.BlockSpec((_BM, D), lambda m: (m, 0)),
        _full(D, 2 * H),
        _full(1, H),
    ],
    out_specs=[pl.BlockSpec((_BM, D), lambda m: (m, 0)),
               pl.BlockSpec((_BM, H), lambda m: (m, 0))],
    out_shape=[jax.ShapeDtypeStruct((N, D), jnp.float32),
               jax.ShapeDtypeStruct((N, H), jnp.float32)],
)

_k2 = pl.pallas_call(
    _k2_body,
    grid=(_GRID,),
    in_specs=[
        pl.BlockSpec((NC, _BM, D), lambda m: (0, m, 0)),
        pl.BlockSpec((_BM, H), lambda m: (m, 0)),
        _full(H, 2 * H),
        _full(1, H),
    ],
    out_specs=[pl.BlockSpec((_BM, D), lambda m: (m, 0))] * 2,
    out_shape=[jax.ShapeDtypeStruct((N, D), jnp.float32)] * 2,
)

_k3 = pl.pallas_call(
    _k3_body,
    grid=(_GRID,),
    in_specs=[
        pl.BlockSpec((NC, _BM, D), lambda m: (0, m, 0)),
        pl.BlockSpec((_BM, D), lambda m: (m, 0)),
        _full(1, H),
        _full(1, 1),
    ],
    out_specs=pl.BlockSpec((_BM, 1), lambda m: (m, 0)),
    out_shape=jax.ShapeDtypeStruct((N, 1), jnp.float32),
)


def kernel(x, edge_index, W1l, W1r, b1, W2l, W2r, b2, Wout, bout):
    pad = E_PAD - edge_index.shape[1]
    ar = jnp.arange(pad, dtype=jnp.int32)
    pad_pk = (ar * 97) % N + (N + ar % (N_ACC - N)) * (1 << SHIFT)
    epk = jnp.concatenate(
        [edge_index[0] + edge_index[1] * (1 << SHIFT), pad_pk]
    ).reshape(NCHUNK, CH)

    w1 = jnp.concatenate([W1l.T, W1r.T], axis=1)
    w2 = jnp.concatenate([W2l.T, W2r.T], axis=1)
    b1r = b1.reshape(1, H)
    b2r = b2.reshape(1, H)
    wrow = Wout.reshape(1, H)
    br = bout.reshape(1, 1)

    t1, xr1 = _k1(x, w1, b1r)
    p1 = _sc_segsum(t1, epk)
    t2, xe2 = _k2(p1, xr1, w2, b2r)
    p2 = _sc_segsum(t2, epk)
    out = _k3(p2, xe2, wrow, br)
    return out[:, 0]


# speedup vs baseline: 1.3695x; 1.3695x over previous
import functools

import jax
import jax.numpy as jnp
from jax import lax
from jax.experimental import pallas as pl
from jax.experimental.pallas import tpu as pltpu
from jax.experimental.pallas import tpu_sc as plsc

N = 10000
D = 128
H = 64

NC = 2
NS = 16
NW = NC * NS
CH = 128
CPT = 80
NCHUNK = NW * CPT
E_PAD = NCHUNK * CH
N_ACC = 10112
RPT = N_ACC // NS
NBUF = 4
SHIFT = 14
MASK = (1 << SHIFT) - 1
L = 16
AW1 = 80

_vmesh = plsc.VectorSubcoreMesh(core_axis_name="c", subcore_axis_name="s")
_sc_params = pltpu.CompilerParams(use_tc_tiling_on_sc=False)


def _make_segsum(AW):
    @functools.partial(
        pl.kernel,
        out_type=jax.ShapeDtypeStruct((NC, N_ACC, AW), jnp.float32),
        mesh=_vmesh,
        scratch_types=[
            pltpu.VMEM((CPT, CH), jnp.int32),
            pltpu.VMEM((NBUF, CH), jnp.int32),
            pltpu.VMEM((NBUF, CH), jnp.int32),
            pltpu.VMEM((NBUF, CH, AW), jnp.float32),
            pltpu.VMEM_SHARED((N_ACC, AW), jnp.float32),
            pltpu.SemaphoreType.DMA((NBUF,)),
            pltpu.SemaphoreType.DMA((NBUF,)),
        ],
        compiler_params=_sc_params,
    )
    def _sc_segsum(table_hbm, epk_hbm, out_hbm,
                   epk, sidx, didx, bufs, acc, gsem, ssem):
        c = lax.axis_index("c")
        s = lax.axis_index("s")
        base_chunk = (c * NS + s) * CPT

        pltpu.sync_copy(epk_hbm.at[pl.ds(base_chunk, CPT)], epk)

        @pl.loop(0, CH)
        def _(i):
            @pl.loop(0, AW // L)
            def _(j):
                bufs[0, i, pl.ds(j * L, L)] = jnp.zeros((L,), jnp.float32)

        @pl.loop(0, RPT // CH)
        def _(j):
            pltpu.sync_copy(bufs.at[0], acc.at[pl.ds(s * RPT + j * CH, CH)])

        pltpu.sync_copy(bufs.at[0, pl.ds(0, RPT % CH)],
                        acc.at[pl.ds(s * RPT + (RPT // CH) * CH, RPT % CH)])

        plsc.subcore_barrier()

        def _unpack(j):
            b = j % NBUF
            for g in range(CH // L):
                v = epk[j, pl.ds(g * L, L)]
                sidx[b, pl.ds(g * L, L)] = jnp.bitwise_and(v, MASK)
                didx[b, pl.ds(g * L, L)] = jnp.right_shift(v, SHIFT)

        def _gather(j):
            b = j % NBUF
            return pltpu.make_async_copy(table_hbm.at[sidx.at[b]],
                                         bufs.at[b], gsem.at[b])

        def _scatter(i):
            b = i % NBUF
            return pltpu.make_async_copy(bufs.at[b], acc.at[didx.at[b]],
                                         ssem.at[b])

        for j in range(2):
            _unpack(j)
            pltpu.async_copy(table_hbm.at[sidx.at[j]], bufs.at[j],
                             gsem.at[j])

        for i in range(CPT):
            b = i % NBUF
            _gather(i).wait()
            pltpu.async_copy(bufs.at[b], acc.at[didx.at[b]], ssem.at[b],
                             add=True)
            j = i + 2
            if j < CPT:
                jb = j % NBUF
                if j >= NBUF:
                    _scatter(j - NBUF).wait()
                _unpack(j)
                pltpu.async_copy(table_hbm.at[sidx.at[jb]], bufs.at[jb],
                                 gsem.at[jb])

        for m in range(CPT - NBUF, CPT):
            _scatter(m).wait()

        plsc.subcore_barrier()

        pltpu.sync_copy(acc.at[pl.ds(s * RPT, RPT)],
                        out_hbm.at[c].at[pl.ds(s * RPT, RPT)])

    return _sc_segsum


_seg1 = _make_segsum(AW1)
_seg2 = _make_segsum(H)


_BM = 1000
_GRID = N // _BM


def _k1_body(x_ref, w_ref, b_ref, t_ref, xr_ref):
    o = jnp.dot(x_ref[...], w_ref[...], preferred_element_type=jnp.float32)
    t_ref[:, :H] = o[:, :H]
    t_ref[:, H:] = jnp.zeros((_BM, AW1 - H), jnp.float32)
    t_ref[:, H:H + 1] = jnp.ones((_BM, 1), jnp.float32)
    xr_ref[...] = o[:, H:] + b_ref[...]


def _k2_body(p_ref, xr_ref, w_ref, b_ref, t_ref, xe_ref):
    psum = p_ref[0] + p_ref[1]
    invd = 1.0 / jnp.maximum(psum[:, H:H + 1], 1.0)
    h = jnp.maximum(psum[:, :H] * invd + xr_ref[...], 0.0)
    o = jnp.dot(h, w_ref[...], preferred_element_type=jnp.float32)
    t_ref[...] = o[:, :H]
    xe_ref[:, :H] = o[:, H:] + b_ref[...]
    xe_ref[:, H:] = jnp.zeros((_BM, AW1 - H), jnp.float32)
    xe_ref[:, H:H + 1] = invd


def _k3_body(q_ref, xe_ref, wrow_ref, b_ref, out_ref):
    qsum = q_ref[0] + q_ref[1]
    invd = xe_ref[:, H:H + 1]
    h = jnp.maximum(qsum * invd + xe_ref[:, :H], 0.0)
    out_ref[...] = (jnp.sum(h * wrow_ref[...], axis=1, keepdims=True)
                    + b_ref[...])


_full = lambda *shape: pl.BlockSpec(shape, lambda m: tuple(0 for _ in shape))

_k1 = pl.pallas_call(
    _k1_body,
    grid=(_GRID,),
    in_specs=[
        pl.BlockSpec((_BM, D), lambda m: (m, 0)),
        _full(D, 2 * H),
        _full(1, H),
    ],
    out_specs=[pl.BlockSpec((_BM, AW1), lambda m: (m, 0)),
               pl.BlockSpec((_BM, H), lambda m: (m, 0))],
    out_shape=[jax.ShapeDtypeStruct((N, AW1), jnp.float32),
               jax.ShapeDtypeStruct((N, H), jnp.float32)],
)

_k2 = pl.pallas_call(
    _k2_body,
    grid=(_GRID,),
    in_specs=[
        pl.BlockSpec((NC, _BM, AW1), lambda m: (0, m, 0)),
        pl.BlockSpec((_BM, H), lambda m: (m, 0)),
        _full(H, 2 * H),
        _full(1, H),
    ],
    out_specs=[pl.BlockSpec((_BM, H), lambda m: (m, 0)),
               pl.BlockSpec((_BM, AW1), lambda m: (m, 0))],
    out_shape=[jax.ShapeDtypeStruct((N, H), jnp.float32),
               jax.ShapeDtypeStruct((N, AW1), jnp.float32)],
)

_k3 = pl.pallas_call(
    _k3_body,
    grid=(_GRID,),
    in_specs=[
        pl.BlockSpec((NC, _BM, H), lambda m: (0, m, 0)),
        pl.BlockSpec((_BM, AW1), lambda m: (m, 0)),
        _full(1, H),
        _full(1, 1),
    ],
    out_specs=pl.BlockSpec((_BM, 1), lambda m: (m, 0)),
    out_shape=jax.ShapeDtypeStruct((N, 1), jnp.float32),
)


def kernel(x, edge_index, W1l, W1r, b1, W2l, W2r, b2, Wout, bout):
    pad = E_PAD - edge_index.shape[1]
    ar = jnp.arange(pad, dtype=jnp.int32)
    pad_pk = (ar * 97) % N + (N + ar % (N_ACC - N)) * (1 << SHIFT)
    epk = jnp.concatenate(
        [edge_index[0] + edge_index[1] * (1 << SHIFT), pad_pk]
    ).reshape(NCHUNK, CH)

    w1 = jnp.concatenate([W1l.T, W1r.T], axis=1)
    w2 = jnp.concatenate([W2l.T, W2r.T], axis=1)
    b1r = b1.reshape(1, H)
    b2r = b2.reshape(1, H)
    wrow = Wout.reshape(1, H)
    br = bout.reshape(1, 1)

    t1, xr1 = _k1(x, w1, b1r)
    p1 = _seg1(t1, epk)
    t2, xe2 = _k2(p1, xr1, w2, b2r)
    p2 = _seg2(t2, epk)
    out = _k3(p2, xe2, wrow, br)
    return out[:, 0]


# speedup vs baseline: 1.4035x; 1.0249x over previous
import functools

import jax
import jax.numpy as jnp
from jax import lax
from jax.experimental import pallas as pl
from jax.experimental.pallas import tpu as pltpu
from jax.experimental.pallas import tpu_sc as plsc

N = 10000
D = 128
H = 64

NC = 2
NS = 16
NW = NC * NS
CH = 128
CPT = 80
NCHUNK = NW * CPT
E_PAD = NCHUNK * CH
N_ACC = 10112
RPT = N_ACC // NS
NBUF = 4
SHIFT = 14
MASK = (1 << SHIFT) - 1
L = 16
AW1 = 80

_vmesh = plsc.VectorSubcoreMesh(core_axis_name="c", subcore_axis_name="s")
_sc_params = pltpu.CompilerParams(use_tc_tiling_on_sc=False)


def _make_segsum(AW):
    @functools.partial(
        pl.kernel,
        out_type=jax.ShapeDtypeStruct((NC, N_ACC, AW), jnp.float32),
        mesh=_vmesh,
        scratch_types=[
            pltpu.VMEM((CPT, CH), jnp.int32),
            pltpu.VMEM((NBUF, CH), jnp.int32),
            pltpu.VMEM((NBUF, CH), jnp.int32),
            pltpu.VMEM((NBUF, CH, AW), jnp.float32),
            pltpu.VMEM_SHARED((N_ACC, AW), jnp.float32),
            pltpu.SemaphoreType.DMA((NBUF,)),
            pltpu.SemaphoreType.DMA((NBUF,)),
        ],
        compiler_params=_sc_params,
    )
    def _sc_segsum(table_hbm, epk_hbm, out_hbm,
                   epk, sidx, didx, bufs, acc, gsem, ssem):
        c = lax.axis_index("c")
        s = lax.axis_index("s")
        base_chunk = (c * NS + s) * CPT

        pltpu.sync_copy(epk_hbm.at[pl.ds(base_chunk, CPT)], epk)

        @pl.loop(0, CH)
        def _(i):
            @pl.loop(0, AW // L)
            def _(j):
                bufs[0, i, pl.ds(j * L, L)] = jnp.zeros((L,), jnp.float32)

        @pl.loop(0, RPT // CH)
        def _(j):
            pltpu.sync_copy(bufs.at[0], acc.at[pl.ds(s * RPT + j * CH, CH)])

        pltpu.sync_copy(bufs.at[0, pl.ds(0, RPT % CH)],
                        acc.at[pl.ds(s * RPT + (RPT // CH) * CH, RPT % CH)])

        plsc.subcore_barrier()

        def _unpack(j):
            b = j % NBUF
            for g in range(CH // L):
                v = epk[j, pl.ds(g * L, L)]
                sidx[b, pl.ds(g * L, L)] = jnp.bitwise_and(v, MASK)
                didx[b, pl.ds(g * L, L)] = jnp.right_shift(v, SHIFT)

        def _gather(j):
            b = j % NBUF
            return pltpu.make_async_copy(table_hbm.at[sidx.at[b]],
                                         bufs.at[b], gsem.at[b])

        def _scatter(i):
            b = i % NBUF
            return pltpu.make_async_copy(bufs.at[b], acc.at[didx.at[b]],
                                         ssem.at[b])

        for j in range(2):
            _unpack(j)
            pltpu.async_copy(table_hbm.at[sidx.at[j]], bufs.at[j],
                             gsem.at[j])

        for i in range(CPT):
            b = i % NBUF
            _gather(i).wait()
            pltpu.async_copy(bufs.at[b], acc.at[didx.at[b]], ssem.at[b],
                             add=True)
            j = i + 2
            if j < CPT:
                jb = j % NBUF
                if j >= NBUF:
                    _scatter(j - NBUF).wait()
                _unpack(j)
                pltpu.async_copy(table_hbm.at[sidx.at[jb]], bufs.at[jb],
                                 gsem.at[jb])

        for m in range(CPT - NBUF, CPT):
            _scatter(m).wait()

        plsc.subcore_barrier()

        pltpu.sync_copy(acc.at[pl.ds(s * RPT, RPT)],
                        out_hbm.at[c].at[pl.ds(s * RPT, RPT)])

    return _sc_segsum


_seg1 = _make_segsum(AW1)
_seg2 = _make_segsum(H)


_BM = 2000
_GRID = N // _BM


def _dot_nt(a, w):
    return jax.lax.dot_general(a, w, (((1,), (1,)), ((), ())),
                               preferred_element_type=jnp.float32)


def _k1_body(x_ref, w_ref, b_ref, t_ref, xr_ref):
    o = _dot_nt(x_ref[...], w_ref[...])
    t_ref[:, :H] = o[:, :H]
    t_ref[:, H:] = jnp.zeros((_BM, AW1 - H), jnp.float32)
    t_ref[:, H:H + 1] = jnp.ones((_BM, 1), jnp.float32)
    xr_ref[...] = o[:, H:] + b_ref[...]


def _k2_body(p_ref, xr_ref, w_ref, b_ref, t_ref, xe_ref):
    psum = p_ref[0] + p_ref[1]
    invd = 1.0 / jnp.maximum(psum[:, H:H + 1], 1.0)
    h = jnp.maximum(psum[:, :H] * invd + xr_ref[...], 0.0)
    o = _dot_nt(h, w_ref[...])
    t_ref[...] = o[:, :H]
    xe_ref[:, :H] = o[:, H:] + b_ref[...]
    xe_ref[:, H:] = jnp.zeros((_BM, AW1 - H), jnp.float32)
    xe_ref[:, H:H + 1] = invd


def _k3_body(q_ref, xe_ref, wrow_ref, b_ref, out_ref):
    qsum = q_ref[0] + q_ref[1]
    invd = xe_ref[:, H:H + 1]
    h = jnp.maximum(qsum * invd + xe_ref[:, :H], 0.0)
    out_ref[...] = (jnp.sum(h * wrow_ref[...], axis=1, keepdims=True)
                    + b_ref[...])


_full = lambda *shape: pl.BlockSpec(shape, lambda m: tuple(0 for _ in shape))

_k1 = pl.pallas_call(
    _k1_body,
    grid=(_GRID,),
    in_specs=[
        pl.BlockSpec((_BM, D), lambda m: (m, 0)),
        _full(D, 2 * H),
        _full(1, H),
    ],
    out_specs=[pl.BlockSpec((_BM, AW1), lambda m: (m, 0)),
               pl.BlockSpec((_BM, H), lambda m: (m, 0))],
    out_shape=[jax.ShapeDtypeStruct((N, AW1), jnp.float32),
               jax.ShapeDtypeStruct((N, H), jnp.float32)],
)

_k2 = pl.pallas_call(
    _k2_body,
    grid=(_GRID,),
    in_specs=[
        pl.BlockSpec((NC, _BM, AW1), lambda m: (0, m, 0)),
        pl.BlockSpec((_BM, H), lambda m: (m, 0)),
        _full(2 * H, H),
        _full(1, H),
    ],
    out_specs=[pl.BlockSpec((_BM, H), lambda m: (m, 0)),
               pl.BlockSpec((_BM, AW1), lambda m: (m, 0))],
    out_shape=[jax.ShapeDtypeStruct((N, H), jnp.float32),
               jax.ShapeDtypeStruct((N, AW1), jnp.float32)],
)

_k3 = pl.pallas_call(
    _k3_body,
    grid=(_GRID,),
    in_specs=[
        pl.BlockSpec((NC, _BM, H), lambda m: (0, m, 0)),
        pl.BlockSpec((_BM, AW1), lambda m: (m, 0)),
        _full(1, H),
        _full(1, 1),
    ],
    out_specs=pl.BlockSpec((_BM, 1), lambda m: (m, 0)),
    out_shape=jax.ShapeDtypeStruct((N, 1), jnp.float32),
)


def kernel(x, edge_index, W1l, W1r, b1, W2l, W2r, b2, Wout, bout):
    pad = E_PAD - edge_index.shape[1]
    ar = jnp.arange(pad, dtype=jnp.int32)
    pad_pk = (ar * 97) % N + (N + ar % (N_ACC - N)) * (1 << SHIFT)
    epk = jnp.concatenate(
        [edge_index[0] + edge_index[1] * (1 << SHIFT), pad_pk]
    ).reshape(NCHUNK, CH)
    x, epk = jax.lax.optimization_barrier((x, epk))

    w1 = jnp.concatenate([W1l, W1r], axis=0)
    w2 = jnp.concatenate([W2l, W2r], axis=0)
    b1r = b1.reshape(1, H)
    b2r = b2.reshape(1, H)
    wrow = Wout.reshape(1, H)
    br = bout.reshape(1, 1)

    t1, xr1 = _k1(x, w1, b1r)
    p1 = _seg1(t1, epk)
    t2, xe2 = _k2(p1, xr1, w2, b2r)
    p2 = _seg2(t2, epk)
    return _k3(p2, xe2, wrow, br)[:, 0]


# speedup vs baseline: 1.4391x; 1.0253x over previous
import functools

import jax
import jax.numpy as jnp
from jax import lax
from jax.experimental import pallas as pl
from jax.experimental.pallas import tpu as pltpu
from jax.experimental.pallas import tpu_sc as plsc

N = 10000
D = 128
H = 64

NC = 2
NS = 16
NW = NC * NS
CH = 128
CPT = 80
NCHUNK = NW * CPT
E_PAD = NCHUNK * CH
N_ACC = 10112
RPT = N_ACC // NS
NRC = 2500
NPC = NCHUNK - NRC
NBUF = 4
SHIFT = 14
MASK = (1 << SHIFT) - 1
L = 16
AW1 = 80

_vmesh = plsc.VectorSubcoreMesh(core_axis_name="c", subcore_axis_name="s")
_sc_params = pltpu.CompilerParams(use_tc_tiling_on_sc=False)


def _make_segsum(AW):
    @functools.partial(
        pl.kernel,
        out_type=jax.ShapeDtypeStruct((NC, N_ACC, AW), jnp.float32),
        mesh=_vmesh,
        scratch_types=[
            pltpu.VMEM((CPT, CH), jnp.int32),
            pltpu.VMEM((NBUF, CH), jnp.int32),
            pltpu.VMEM((NBUF, CH), jnp.int32),
            pltpu.VMEM((NBUF, CH, AW), jnp.float32),
            pltpu.VMEM_SHARED((N_ACC, AW), jnp.float32),
            pltpu.SemaphoreType.DMA((NBUF,)),
            pltpu.SemaphoreType.DMA((NBUF,)),
        ],
        compiler_params=_sc_params,
    )
    def _sc_segsum(table_hbm, epk_hbm, pad_hbm, out_hbm,
                   epk, sidx, didx, bufs, acc, gsem, ssem):
        c = lax.axis_index("c")
        s = lax.axis_index("s")
        w = c * NS + s
        base_chunk = w * CPT

        @pl.when(w < NW - 1)
        def _():
            pltpu.sync_copy(epk_hbm.at[pl.ds(base_chunk, CPT)], epk)

        @pl.when(w == NW - 1)
        def _():
            pltpu.sync_copy(epk_hbm.at[pl.ds(NRC - (CPT - NPC), CPT - NPC)],
                            epk.at[pl.ds(0, CPT - NPC)])
            pltpu.sync_copy(pad_hbm, epk.at[pl.ds(CPT - NPC, NPC)])

        @pl.loop(0, CH)
        def _(i):
            @pl.loop(0, AW // L)
            def _(j):
                bufs[0, i, pl.ds(j * L, L)] = jnp.zeros((L,), jnp.float32)

        @pl.loop(0, RPT // CH)
        def _(j):
            pltpu.sync_copy(bufs.at[0], acc.at[pl.ds(s * RPT + j * CH, CH)])

        pltpu.sync_copy(bufs.at[0, pl.ds(0, RPT % CH)],
                        acc.at[pl.ds(s * RPT + (RPT // CH) * CH, RPT % CH)])

        plsc.subcore_barrier()

        def _unpack(j):
            b = j % NBUF
            for g in range(CH // L):
                v = epk[j, pl.ds(g * L, L)]
                sidx[b, pl.ds(g * L, L)] = jnp.bitwise_and(v, MASK)
                didx[b, pl.ds(g * L, L)] = jnp.right_shift(v, SHIFT)

        def _gather(j):
            b = j % NBUF
            return pltpu.make_async_copy(table_hbm.at[sidx.at[b]],
                                         bufs.at[b], gsem.at[b])

        def _scatter(i):
            b = i % NBUF
            return pltpu.make_async_copy(bufs.at[b], acc.at[didx.at[b]],
                                         ssem.at[b])

        for j in range(2):
            _unpack(j)
            pltpu.async_copy(table_hbm.at[sidx.at[j]], bufs.at[j],
                             gsem.at[j])

        for i in range(CPT):
            b = i % NBUF
            _gather(i).wait()
            pltpu.async_copy(bufs.at[b], acc.at[didx.at[b]], ssem.at[b],
                             add=True)
            j = i + 2
            if j < CPT:
                jb = j % NBUF
                if j >= NBUF:
                    _scatter(j - NBUF).wait()
                _unpack(j)
                pltpu.async_copy(table_hbm.at[sidx.at[jb]], bufs.at[jb],
                                 gsem.at[jb])

        for m in range(CPT - NBUF, CPT):
            _scatter(m).wait()

        plsc.subcore_barrier()

        pltpu.sync_copy(acc.at[pl.ds(s * RPT, RPT)],
                        out_hbm.at[c].at[pl.ds(s * RPT, RPT)])

    return _sc_segsum


_seg1 = _make_segsum(AW1)
_seg2 = _make_segsum(H)


_BM = 2000
_GRID = N // _BM


def _dot_nt(a, w):
    return jax.lax.dot_general(a, w, (((1,), (1,)), ((), ())),
                               preferred_element_type=jnp.float32)


def _k1_body(x_ref, w_ref, b_ref, t_ref, xr_ref):
    o = _dot_nt(x_ref[...], w_ref[...])
    t_ref[:, :H] = o[:, :H]
    t_ref[:, H:] = jnp.zeros((_BM, AW1 - H), jnp.float32)
    t_ref[:, H:H + 1] = jnp.ones((_BM, 1), jnp.float32)
    xr_ref[...] = o[:, H:] + b_ref[...]


def _k2_body(p_ref, xr_ref, w_ref, b_ref, t_ref, xe_ref):
    psum = p_ref[0] + p_ref[1]
    invd = 1.0 / jnp.maximum(psum[:, H:H + 1], 1.0)
    h = jnp.maximum(psum[:, :H] * invd + xr_ref[...], 0.0)
    o = _dot_nt(h, w_ref[...])
    t_ref[...] = o[:, :H]
    xe_ref[:, :H] = o[:, H:] + b_ref[...]
    xe_ref[:, H:] = jnp.zeros((_BM, AW1 - H), jnp.float32)
    xe_ref[:, H:H + 1] = invd


def _k3_body(q_ref, xe_ref, wrow_ref, b_ref, out_ref):
    qsum = q_ref[0] + q_ref[1]
    invd = xe_ref[:, H:H + 1]
    h = jnp.maximum(qsum * invd + xe_ref[:, :H], 0.0)
    out_ref[0] = _dot_nt(wrow_ref[...], h) + b_ref[...]


_full = lambda *shape: pl.BlockSpec(shape, lambda m: tuple(0 for _ in shape))

_k1 = pl.pallas_call(
    _k1_body,
    grid=(_GRID,),
    in_specs=[
        pl.BlockSpec((_BM, D), lambda m: (m, 0)),
        _full(D, 2 * H),
        _full(1, H),
    ],
    out_specs=[pl.BlockSpec((_BM, AW1), lambda m: (m, 0)),
               pl.BlockSpec((_BM, H), lambda m: (m, 0))],
    out_shape=[jax.ShapeDtypeStruct((N, AW1), jnp.float32),
               jax.ShapeDtypeStruct((N, H), jnp.float32)],
)

_k2 = pl.pallas_call(
    _k2_body,
    grid=(_GRID,),
    in_specs=[
        pl.BlockSpec((NC, _BM, AW1), lambda m: (0, m, 0)),
        pl.BlockSpec((_BM, H), lambda m: (m, 0)),
        _full(2 * H, H),
        _full(1, H),
    ],
    out_specs=[pl.BlockSpec((_BM, H), lambda m: (m, 0)),
               pl.BlockSpec((_BM, AW1), lambda m: (m, 0))],
    out_shape=[jax.ShapeDtypeStruct((N, H), jnp.float32),
               jax.ShapeDtypeStruct((N, AW1), jnp.float32)],
)

_k3 = pl.pallas_call(
    _k3_body,
    grid=(_GRID,),
    in_specs=[
        pl.BlockSpec((NC, _BM, H), lambda m: (0, m, 0)),
        pl.BlockSpec((_BM, AW1), lambda m: (m, 0)),
        _full(1, H),
        _full(1, 1),
    ],
    out_specs=pl.BlockSpec((1, 1, _BM), lambda m: (m, 0, 0)),
    out_shape=jax.ShapeDtypeStruct((_GRID, 1, _BM), jnp.float32),
)


def kernel(x, edge_index, W1l, W1r, b1, W2l, W2r, b2, Wout, bout):
    epk = (edge_index[0] + edge_index[1] * (1 << SHIFT)).reshape(NRC, CH)
    ar = jnp.arange(NPC * CH, dtype=jnp.int32)
    pad_pk = ((ar * 97) % N
              + (N + ar % (N_ACC - N)) * (1 << SHIFT)).reshape(NPC, CH)
    x, epk = jax.lax.optimization_barrier((x, epk))

    w1 = jnp.concatenate([W1l, W1r], axis=0)
    w2 = jnp.concatenate([W2l, W2r], axis=0)
    b1r = b1.reshape(1, H)
    b2r = b2.reshape(1, H)
    wrow = Wout.reshape(1, H)
    br = bout.reshape(1, 1)

    t1, xr1 = _k1(x, w1, b1r)
    p1 = _seg1(t1, epk, pad_pk)
    t2, xe2 = _k2(p1, xr1, w2, b2r)
    p2 = _seg2(t2, epk, pad_pk)
    return _k3(p2, xe2, wrow, br).reshape(N)


# speedup vs baseline: 1.5380x; 1.0687x over previous
import functools

import jax
import jax.numpy as jnp
from jax import lax
from jax.experimental import pallas as pl
from jax.experimental.pallas import tpu as pltpu
from jax.experimental.pallas import tpu_sc as plsc

N = 10000
D = 128
H = 64

NC = 2
NS = 16
NW = NC * NS
CH = 128
CPT = 80
NCHUNK = NW * CPT
E_PAD = NCHUNK * CH
N_ACC = 10112
RPT = N_ACC // NS
NRC = 2500
NPC = NCHUNK - NRC
NBUF = 4
SHIFT = 14
MASK = (1 << SHIFT) - 1
L = 16
AW1 = 80

_vmesh = plsc.VectorSubcoreMesh(core_axis_name="c", subcore_axis_name="s")
_sc_params = pltpu.CompilerParams(use_tc_tiling_on_sc=False)


def _make_segsum(AW):
    @functools.partial(
        pl.kernel,
        out_type=jax.ShapeDtypeStruct((NC, N_ACC, AW), jnp.float32),
        mesh=_vmesh,
        scratch_types=[
            pltpu.VMEM((CPT, CH), jnp.int32),
            pltpu.VMEM((NBUF, CH), jnp.int32),
            pltpu.VMEM((NBUF, CH), jnp.int32),
            pltpu.VMEM((NBUF, CH, AW), jnp.float32),
            pltpu.VMEM_SHARED((N_ACC, AW), jnp.float32),
            pltpu.SemaphoreType.DMA((NBUF,)),
            pltpu.SemaphoreType.DMA((NBUF,)),
        ],
        compiler_params=_sc_params,
    )
    def _sc_segsum(table_hbm, epk_hbm, pad_hbm, out_hbm,
                   epk, sidx, didx, bufs, acc, gsem, ssem):
        c = lax.axis_index("c")
        s = lax.axis_index("s")
        w = c * NS + s
        base_chunk = w * CPT

        @pl.when(w < NW - 1)
        def _():
            pltpu.sync_copy(epk_hbm.at[pl.ds(base_chunk, CPT)], epk)

        @pl.when(w == NW - 1)
        def _():
            pltpu.sync_copy(epk_hbm.at[pl.ds(NRC - (CPT - NPC), CPT - NPC)],
                            epk.at[pl.ds(0, CPT - NPC)])
            pltpu.sync_copy(pad_hbm, epk.at[pl.ds(CPT - NPC, NPC)])

        @pl.loop(0, CH)
        def _(i):
            @pl.loop(0, AW // L)
            def _(j):
                bufs[0, i, pl.ds(j * L, L)] = jnp.zeros((L,), jnp.float32)

        @pl.loop(0, RPT // CH)
        def _(j):
            pltpu.sync_copy(bufs.at[0], acc.at[pl.ds(s * RPT + j * CH, CH)])

        pltpu.sync_copy(bufs.at[0, pl.ds(0, RPT % CH)],
                        acc.at[pl.ds(s * RPT + (RPT // CH) * CH, RPT % CH)])

        plsc.subcore_barrier()

        def _unpack(j):
            b = j % NBUF
            for g in range(CH // L):
                v = epk[j, pl.ds(g * L, L)]
                sidx[b, pl.ds(g * L, L)] = jnp.bitwise_and(v, MASK)
                didx[b, pl.ds(g * L, L)] = jnp.right_shift(v, SHIFT)

        def _gather(j):
            b = j % NBUF
            return pltpu.make_async_copy(table_hbm.at[sidx.at[b]],
                                         bufs.at[b], gsem.at[b])

        def _scatter(i):
            b = i % NBUF
            return pltpu.make_async_copy(bufs.at[b], acc.at[didx.at[b]],
                                         ssem.at[b])

        for j in range(3):
            _unpack(j)
            pltpu.async_copy(table_hbm.at[sidx.at[j]], bufs.at[j],
                             gsem.at[j])

        for i in range(CPT):
            b = i % NBUF
            _gather(i).wait()
            pltpu.async_copy(bufs.at[b], acc.at[didx.at[b]], ssem.at[b],
                             add=True)
            j = i + 3
            if j < CPT:
                jb = j % NBUF
                if j >= NBUF:
                    _scatter(j - NBUF).wait()
                _unpack(j)
                pltpu.async_copy(table_hbm.at[sidx.at[jb]], bufs.at[jb],
                                 gsem.at[jb])

        for m in range(CPT - NBUF, CPT):
            _scatter(m).wait()

        plsc.subcore_barrier()

        pltpu.sync_copy(acc.at[pl.ds(s * RPT, RPT)],
                        out_hbm.at[c].at[pl.ds(s * RPT, RPT)])

    return _sc_segsum


_seg1 = _make_segsum(AW1)
_seg2 = _make_segsum(H)


_BM = 2000
_GRID = N // _BM


def _dot_nt(a, w):
    return jax.lax.dot_general(a, w, (((1,), (1,)), ((), ())),
                               preferred_element_type=jnp.float32)


def _k1_body(x_ref, w_ref, b_ref, t_ref, xr_ref):
    o = _dot_nt(x_ref[...], w_ref[...])
    t_ref[:, :H] = o[:, :H]
    t_ref[:, H:] = jnp.zeros((_BM, AW1 - H), jnp.float32)
    t_ref[:, H:H + 1] = jnp.ones((_BM, 1), jnp.float32)
    xr_ref[...] = o[:, H:] + b_ref[...]


def _k2_body(p_ref, xr_ref, w_ref, b_ref, t_ref, xe_ref):
    psum = p_ref[0] + p_ref[1]
    invd = 1.0 / jnp.maximum(psum[:, H:H + 1], 1.0)
    h = jnp.maximum(psum[:, :H] * invd + xr_ref[...], 0.0)
    o = _dot_nt(h, w_ref[...])
    t_ref[...] = o[:, :H]
    xe_ref[:, :H] = o[:, H:] + b_ref[...]
    xe_ref[:, H:] = jnp.zeros((_BM, AW1 - H), jnp.float32)
    xe_ref[:, H:H + 1] = invd


def _k3_body(q_ref, xe_ref, wrow_ref, b_ref, out_ref):
    qsum = q_ref[0] + q_ref[1]
    invd = xe_ref[:, H:H + 1]
    h = jnp.maximum(qsum * invd + xe_ref[:, :H], 0.0)
    out_ref[0] = _dot_nt(wrow_ref[...], h) + b_ref[...]


_full = lambda *shape: pl.BlockSpec(shape, lambda m: tuple(0 for _ in shape))

_k1 = pl.pallas_call(
    _k1_body,
    grid=(_GRID,),
    in_specs=[
        pl.BlockSpec((_BM, D), lambda m: (m, 0)),
        _full(D, 2 * H),
        _full(1, H),
    ],
    out_specs=[pl.BlockSpec((_BM, AW1), lambda m: (m, 0)),
               pl.BlockSpec((_BM, H), lambda m: (m, 0))],
    out_shape=[jax.ShapeDtypeStruct((N, AW1), jnp.float32),
               jax.ShapeDtypeStruct((N, H), jnp.float32)],
)

_k2 = pl.pallas_call(
    _k2_body,
    grid=(_GRID,),
    in_specs=[
        pl.BlockSpec((NC, _BM, AW1), lambda m: (0, m, 0)),
        pl.BlockSpec((_BM, H), lambda m: (m, 0)),
        _full(2 * H, H),
        _full(1, H),
    ],
    out_specs=[pl.BlockSpec((_BM, H), lambda m: (m, 0)),
               pl.BlockSpec((_BM, AW1), lambda m: (m, 0))],
    out_shape=[jax.ShapeDtypeStruct((N, H), jnp.float32),
               jax.ShapeDtypeStruct((N, AW1), jnp.float32)],
)

_k3 = pl.pallas_call(
    _k3_body,
    grid=(_GRID,),
    in_specs=[
        pl.BlockSpec((NC, _BM, H), lambda m: (0, m, 0)),
        pl.BlockSpec((_BM, AW1), lambda m: (m, 0)),
        _full(1, H),
        _full(1, 1),
    ],
    out_specs=pl.BlockSpec((1, 1, _BM), lambda m: (m, 0, 0)),
    out_shape=jax.ShapeDtypeStruct((_GRID, 1, _BM), jnp.float32),
)


def kernel(x, edge_index, W1l, W1r, b1, W2l, W2r, b2, Wout, bout):
    epk = (edge_index[0] + edge_index[1] * (1 << SHIFT)).reshape(NRC, CH)
    ar = jnp.arange(NPC * CH, dtype=jnp.int32)
    pad_pk = ((ar * 97) % N
              + (N + ar % (N_ACC - N)) * (1 << SHIFT)).reshape(NPC, CH)
    x, epk = jax.lax.optimization_barrier((x, epk))

    w1 = jnp.concatenate([W1l, W1r], axis=0)
    w2 = jnp.concatenate([W2l, W2r], axis=0)
    b1r = b1.reshape(1, H)
    b2r = b2.reshape(1, H)
    wrow = Wout.reshape(1, H)
    br = bout.reshape(1, 1)

    t1, xr1 = _k1(x, w1, b1r)
    p1 = _seg1(t1, epk, pad_pk)
    t2, xe2 = _k2(p1, xr1, w2, b2r)
    p2 = _seg2(t2, epk, pad_pk)
    return _k3(p2, xe2, wrow, br).reshape(N)


# speedup vs baseline: 1.5900x; 1.0338x over previous
import functools

import jax
import jax.numpy as jnp
from jax import lax
from jax.experimental import pallas as pl
from jax.experimental.pallas import tpu as pltpu
from jax.experimental.pallas import tpu_sc as plsc

N = 10000
D = 128
H = 64

NC = 2
NS = 16
NW = NC * NS
CH = 128
CPT = 80
NCHUNK = NW * CPT
E_PAD = NCHUNK * CH
N_ACC = 10112
RPT = N_ACC // NS
NRC = 2500
NPC = NCHUNK - NRC
NBUF = 5
SHIFT = 14
MASK = (1 << SHIFT) - 1
L = 16
AW1 = 80

_vmesh = plsc.VectorSubcoreMesh(core_axis_name="c", subcore_axis_name="s")
_sc_params = pltpu.CompilerParams(use_tc_tiling_on_sc=False)


def _make_segsum(AW):
    @functools.partial(
        pl.kernel,
        out_type=jax.ShapeDtypeStruct((NC, N_ACC, AW), jnp.float32),
        mesh=_vmesh,
        scratch_types=[
            pltpu.VMEM((CPT, CH), jnp.int32),
            pltpu.VMEM((NBUF, CH), jnp.int32),
            pltpu.VMEM((NBUF, CH), jnp.int32),
            pltpu.VMEM((NBUF, CH, AW), jnp.float32),
            pltpu.VMEM_SHARED((N_ACC, AW), jnp.float32),
            pltpu.SemaphoreType.DMA((NBUF,)),
            pltpu.SemaphoreType.DMA((NBUF,)),
        ],
        compiler_params=_sc_params,
    )
    def _sc_segsum(table_hbm, epk_hbm, pad_hbm, out_hbm,
                   epk, sidx, didx, bufs, acc, gsem, ssem):
        c = lax.axis_index("c")
        s = lax.axis_index("s")
        w = c * NS + s
        base_chunk = w * CPT

        @pl.when(w < NW - 1)
        def _():
            pltpu.sync_copy(epk_hbm.at[pl.ds(base_chunk, CPT)], epk)

        @pl.when(w == NW - 1)
        def _():
            pltpu.sync_copy(epk_hbm.at[pl.ds(NRC - (CPT - NPC), CPT - NPC)],
                            epk.at[pl.ds(0, CPT - NPC)])
            pltpu.sync_copy(pad_hbm, epk.at[pl.ds(CPT - NPC, NPC)])

        @pl.loop(0, CH)
        def _(i):
            @pl.loop(0, AW // L)
            def _(j):
                bufs[0, i, pl.ds(j * L, L)] = jnp.zeros((L,), jnp.float32)

        @pl.loop(0, RPT // CH)
        def _(j):
            pltpu.sync_copy(bufs.at[0], acc.at[pl.ds(s * RPT + j * CH, CH)])

        pltpu.sync_copy(bufs.at[0, pl.ds(0, RPT % CH)],
                        acc.at[pl.ds(s * RPT + (RPT // CH) * CH, RPT % CH)])

        plsc.subcore_barrier()

        def _unpack(j):
            b = j % NBUF
            for g in range(CH // L):
                v = epk[j, pl.ds(g * L, L)]
                sidx[b, pl.ds(g * L, L)] = jnp.bitwise_and(v, MASK)
                didx[b, pl.ds(g * L, L)] = jnp.right_shift(v, SHIFT)

        def _gather(j):
            b = j % NBUF
            return pltpu.make_async_copy(table_hbm.at[sidx.at[b]],
                                         bufs.at[b], gsem.at[b])

        def _scatter(i):
            b = i % NBUF
            return pltpu.make_async_copy(bufs.at[b], acc.at[didx.at[b]],
                                         ssem.at[b])

        for j in range(4):
            _unpack(j)
            pltpu.async_copy(table_hbm.at[sidx.at[j]], bufs.at[j],
                             gsem.at[j])

        for i in range(CPT):
            b = i % NBUF
            _gather(i).wait()
            pltpu.async_copy(bufs.at[b], acc.at[didx.at[b]], ssem.at[b],
                             add=True)
            j = i + 4
            if j < CPT:
                jb = j % NBUF
                if j >= NBUF:
                    _scatter(j - NBUF).wait()
                _unpack(j)
                pltpu.async_copy(table_hbm.at[sidx.at[jb]], bufs.at[jb],
                                 gsem.at[jb])

        for m in range(CPT - NBUF, CPT):
            _scatter(m).wait()

        plsc.subcore_barrier()

        pltpu.sync_copy(acc.at[pl.ds(s * RPT, RPT)],
                        out_hbm.at[c].at[pl.ds(s * RPT, RPT)])

    return _sc_segsum


_seg1 = _make_segsum(AW1)
_seg2 = _make_segsum(H)


_BM = 2000
_GRID = N // _BM


def _dot_nt(a, w):
    return jax.lax.dot_general(a, w, (((1,), (1,)), ((), ())),
                               preferred_element_type=jnp.float32)


def _k1_body(x_ref, w_ref, b_ref, t_ref, xr_ref):
    o = _dot_nt(x_ref[...], w_ref[...])
    t_ref[:, :H] = o[:, :H]
    t_ref[:, H:] = jnp.zeros((_BM, AW1 - H), jnp.float32)
    t_ref[:, H:H + 1] = jnp.ones((_BM, 1), jnp.float32)
    xr_ref[...] = o[:, H:] + b_ref[...]


def _k2_body(p_ref, xr_ref, w_ref, b_ref, t_ref, xe_ref):
    psum = p_ref[0] + p_ref[1]
    invd = 1.0 / jnp.maximum(psum[:, H:H + 1], 1.0)
    h = jnp.maximum(psum[:, :H] * invd + xr_ref[...], 0.0)
    o = _dot_nt(h, w_ref[...])
    t_ref[...] = o[:, :H]
    xe_ref[:, :H] = o[:, H:] + b_ref[...]
    xe_ref[:, H:] = jnp.zeros((_BM, AW1 - H), jnp.float32)
    xe_ref[:, H:H + 1] = invd


def _k3_body(q_ref, xe_ref, wrow_ref, b_ref, out_ref):
    qsum = q_ref[0] + q_ref[1]
    invd = xe_ref[:, H:H + 1]
    h = jnp.maximum(qsum * invd + xe_ref[:, :H], 0.0)
    out_ref[0] = _dot_nt(wrow_ref[...], h) + b_ref[...]


_full = lambda *shape: pl.BlockSpec(shape, lambda m: tuple(0 for _ in shape))

_k1 = pl.pallas_call(
    _k1_body,
    grid=(_GRID,),
    in_specs=[
        pl.BlockSpec((_BM, D), lambda m: (m, 0)),
        _full(D, 2 * H),
        _full(1, H),
    ],
    out_specs=[pl.BlockSpec((_BM, AW1), lambda m: (m, 0)),
               pl.BlockSpec((_BM, H), lambda m: (m, 0))],
    out_shape=[jax.ShapeDtypeStruct((N, AW1), jnp.float32),
               jax.ShapeDtypeStruct((N, H), jnp.float32)],
)

_k2 = pl.pallas_call(
    _k2_body,
    grid=(_GRID,),
    in_specs=[
        pl.BlockSpec((NC, _BM, AW1), lambda m: (0, m, 0)),
        pl.BlockSpec((_BM, H), lambda m: (m, 0)),
        _full(2 * H, H),
        _full(1, H),
    ],
    out_specs=[pl.BlockSpec((_BM, H), lambda m: (m, 0)),
               pl.BlockSpec((_BM, AW1), lambda m: (m, 0))],
    out_shape=[jax.ShapeDtypeStruct((N, H), jnp.float32),
               jax.ShapeDtypeStruct((N, AW1), jnp.float32)],
)

_k3 = pl.pallas_call(
    _k3_body,
    grid=(_GRID,),
    in_specs=[
        pl.BlockSpec((NC, _BM, H), lambda m: (0, m, 0)),
        pl.BlockSpec((_BM, AW1), lambda m: (m, 0)),
        _full(1, H),
        _full(1, 1),
    ],
    out_specs=pl.BlockSpec((1, 1, _BM), lambda m: (m, 0, 0)),
    out_shape=jax.ShapeDtypeStruct((_GRID, 1, _BM), jnp.float32),
)


def kernel(x, edge_index, W1l, W1r, b1, W2l, W2r, b2, Wout, bout):
    epk = (edge_index[0] + edge_index[1] * (1 << SHIFT)).reshape(NRC, CH)
    ar = jnp.arange(NPC * CH, dtype=jnp.int32)
    pad_pk = ((ar * 97) % N
              + (N + ar % (N_ACC - N)) * (1 << SHIFT)).reshape(NPC, CH)
    x, epk = jax.lax.optimization_barrier((x, epk))

    w1 = jnp.concatenate([W1l, W1r], axis=0)
    w2 = jnp.concatenate([W2l, W2r], axis=0)
    b1r = b1.reshape(1, H)
    b2r = b2.reshape(1, H)
    wrow = Wout.reshape(1, H)
    br = bout.reshape(1, 1)

    t1, xr1 = _k1(x, w1, b1r)
    p1 = _seg1(t1, epk, pad_pk)
    t2, xe2 = _k2(p1, xr1, w2, b2r)
    p2 = _seg2(t2, epk, pad_pk)
    return _k3(p2, xe2, wrow, br).reshape(N)


# speedup vs baseline: 1.6111x; 1.0133x over previous
import functools

import jax
import jax.numpy as jnp
from jax import lax
from jax.experimental import pallas as pl
from jax.experimental.pallas import tpu as pltpu
from jax.experimental.pallas import tpu_sc as plsc

N = 10000
D = 128
H = 64

NC = 2
NS = 16
NW = NC * NS
CH = 128
CPT = 80
NCHUNK = NW * CPT
E_PAD = NCHUNK * CH
N_ACC = 10112
RPT = N_ACC // NS
NRC = 2500
NPC = NCHUNK - NRC
NBUF = 6
SHIFT = 14
MASK = (1 << SHIFT) - 1
L = 16
AW1 = 80

_vmesh = plsc.VectorSubcoreMesh(core_axis_name="c", subcore_axis_name="s")
_sc_params = pltpu.CompilerParams(use_tc_tiling_on_sc=False)


def _make_segsum(AW):
    @functools.partial(
        pl.kernel,
        out_type=jax.ShapeDtypeStruct((NC, N_ACC, AW), jnp.float32),
        mesh=_vmesh,
        scratch_types=[
            pltpu.VMEM((CPT, CH), jnp.int32),
            pltpu.VMEM((NBUF, CH), jnp.int32),
            pltpu.VMEM((NBUF, CH), jnp.int32),
            pltpu.VMEM((NBUF, CH, AW), jnp.float32),
            pltpu.VMEM_SHARED((N_ACC, AW), jnp.float32),
            pltpu.SemaphoreType.DMA((NBUF,)),
            pltpu.SemaphoreType.DMA((NBUF,)),
        ],
        compiler_params=_sc_params,
    )
    def _sc_segsum(table_hbm, epk_hbm, pad_hbm, out_hbm,
                   epk, sidx, didx, bufs, acc, gsem, ssem):
        c = lax.axis_index("c")
        s = lax.axis_index("s")
        w = c * NS + s
        base_chunk = w * CPT

        @pl.when(w < NW - 1)
        def _():
            pltpu.sync_copy(epk_hbm.at[pl.ds(base_chunk, CPT)], epk)

        @pl.when(w == NW - 1)
        def _():
            pltpu.sync_copy(epk_hbm.at[pl.ds(NRC - (CPT - NPC), CPT - NPC)],
                            epk.at[pl.ds(0, CPT - NPC)])
            pltpu.sync_copy(pad_hbm, epk.at[pl.ds(CPT - NPC, NPC)])

        @pl.loop(0, CH)
        def _(i):
            @pl.loop(0, AW // L)
            def _(j):
                bufs[0, i, pl.ds(j * L, L)] = jnp.zeros((L,), jnp.float32)

        @pl.loop(0, RPT // CH)
        def _(j):
            pltpu.sync_copy(bufs.at[0], acc.at[pl.ds(s * RPT + j * CH, CH)])

        pltpu.sync_copy(bufs.at[0, pl.ds(0, RPT % CH)],
                        acc.at[pl.ds(s * RPT + (RPT // CH) * CH, RPT % CH)])

        plsc.subcore_barrier()

        def _unpack(j):
            b = j % NBUF
            for g in range(CH // L):
                v = epk[j, pl.ds(g * L, L)]
                sidx[b, pl.ds(g * L, L)] = jnp.bitwise_and(v, MASK)
                didx[b, pl.ds(g * L, L)] = jnp.right_shift(v, SHIFT)

        def _gather(j):
            b = j % NBUF
            return pltpu.make_async_copy(table_hbm.at[sidx.at[b]],
                                         bufs.at[b], gsem.at[b])

        def _scatter(i):
            b = i % NBUF
            return pltpu.make_async_copy(bufs.at[b], acc.at[didx.at[b]],
                                         ssem.at[b])

        for j in range(5):
            _unpack(j)
            pltpu.async_copy(table_hbm.at[sidx.at[j]], bufs.at[j],
                             gsem.at[j])

        for i in range(CPT):
            b = i % NBUF
            _gather(i).wait()
            pltpu.async_copy(bufs.at[b], acc.at[didx.at[b]], ssem.at[b],
                             add=True)
            j = i + 5
            if j < CPT:
                jb = j % NBUF
                if j >= NBUF:
                    _scatter(j - NBUF).wait()
                _unpack(j)
                pltpu.async_copy(table_hbm.at[sidx.at[jb]], bufs.at[jb],
                                 gsem.at[jb])

        for m in range(CPT - NBUF, CPT):
            _scatter(m).wait()

        plsc.subcore_barrier()

        pltpu.sync_copy(acc.at[pl.ds(s * RPT, RPT)],
                        out_hbm.at[c].at[pl.ds(s * RPT, RPT)])

    return _sc_segsum


_seg1 = _make_segsum(AW1)
_seg2 = _make_segsum(H)


_BM = 2000
_GRID = N // _BM


def _dot_nt(a, w):
    return jax.lax.dot_general(a, w, (((1,), (1,)), ((), ())),
                               preferred_element_type=jnp.float32)


def _k1_body(x_ref, w_ref, b_ref, t_ref, xr_ref):
    o = _dot_nt(x_ref[...], w_ref[...])
    t_ref[:, :H] = o[:, :H]
    t_ref[:, H:] = jnp.zeros((_BM, AW1 - H), jnp.float32)
    t_ref[:, H:H + 1] = jnp.ones((_BM, 1), jnp.float32)
    xr_ref[...] = o[:, H:] + b_ref[...]


def _k2_body(p_ref, xr_ref, w_ref, b_ref, t_ref, xe_ref):
    psum = p_ref[0] + p_ref[1]
    invd = 1.0 / jnp.maximum(psum[:, H:H + 1], 1.0)
    h = jnp.maximum(psum[:, :H] * invd + xr_ref[...], 0.0)
    o = _dot_nt(h, w_ref[...])
    t_ref[...] = o[:, :H]
    xe_ref[:, :H] = o[:, H:] + b_ref[...]
    xe_ref[:, H:] = jnp.zeros((_BM, AW1 - H), jnp.float32)
    xe_ref[:, H:H + 1] = invd


def _k3_body(q_ref, xe_ref, wrow_ref, b_ref, out_ref):
    qsum = q_ref[0] + q_ref[1]
    invd = xe_ref[:, H:H + 1]
    h = jnp.maximum(qsum * invd + xe_ref[:, :H], 0.0)
    out_ref[0] = _dot_nt(wrow_ref[...], h) + b_ref[...]


_full = lambda *shape: pl.BlockSpec(shape, lambda m: tuple(0 for _ in shape))

_k1 = pl.pallas_call(
    _k1_body,
    grid=(_GRID,),
    in_specs=[
        pl.BlockSpec((_BM, D), lambda m: (m, 0)),
        _full(D, 2 * H),
        _full(1, H),
    ],
    out_specs=[pl.BlockSpec((_BM, AW1), lambda m: (m, 0)),
               pl.BlockSpec((_BM, H), lambda m: (m, 0))],
    out_shape=[jax.ShapeDtypeStruct((N, AW1), jnp.float32),
               jax.ShapeDtypeStruct((N, H), jnp.float32)],
)

_k2 = pl.pallas_call(
    _k2_body,
    grid=(_GRID,),
    in_specs=[
        pl.BlockSpec((NC, _BM, AW1), lambda m: (0, m, 0)),
        pl.BlockSpec((_BM, H), lambda m: (m, 0)),
        _full(2 * H, H),
        _full(1, H),
    ],
    out_specs=[pl.BlockSpec((_BM, H), lambda m: (m, 0)),
               pl.BlockSpec((_BM, AW1), lambda m: (m, 0))],
    out_shape=[jax.ShapeDtypeStruct((N, H), jnp.float32),
               jax.ShapeDtypeStruct((N, AW1), jnp.float32)],
)

_k3 = pl.pallas_call(
    _k3_body,
    grid=(_GRID,),
    in_specs=[
        pl.BlockSpec((NC, _BM, H), lambda m: (0, m, 0)),
        pl.BlockSpec((_BM, AW1), lambda m: (m, 0)),
        _full(1, H),
        _full(1, 1),
    ],
    out_specs=pl.BlockSpec((1, 1, _BM), lambda m: (m, 0, 0)),
    out_shape=jax.ShapeDtypeStruct((_GRID, 1, _BM), jnp.float32),
)


def kernel(x, edge_index, W1l, W1r, b1, W2l, W2r, b2, Wout, bout):
    epk = (edge_index[0] + edge_index[1] * (1 << SHIFT)).reshape(NRC, CH)
    ar = jnp.arange(NPC * CH, dtype=jnp.int32)
    pad_pk = ((ar * 97) % N
              + (N + ar % (N_ACC - N)) * (1 << SHIFT)).reshape(NPC, CH)
    x, epk = jax.lax.optimization_barrier((x, epk))

    w1 = jnp.concatenate([W1l, W1r], axis=0)
    w2 = jnp.concatenate([W2l, W2r], axis=0)
    b1r = b1.reshape(1, H)
    b2r = b2.reshape(1, H)
    wrow = Wout.reshape(1, H)
    br = bout.reshape(1, 1)

    t1, xr1 = _k1(x, w1, b1r)
    p1 = _seg1(t1, epk, pad_pk)
    t2, xe2 = _k2(p1, xr1, w2, b2r)
    p2 = _seg2(t2, epk, pad_pk)
    return _k3(p2, xe2, wrow, br).reshape(N)
